# baseline (speedup 1.0000x reference)
import jax
import jax.numpy as jnp
from jax import lax
from jax.experimental import pallas as pl
from jax.experimental.pallas import tpu as pltpu
from jax.experimental.pallas import tpu_sc as plsc

N = 10000
E = 320000
D = 128
NPAD = 10240
NC = 2
NS = 16
NW = NC * NS
B = 128
KCI = 16
KC = 8
K = KCI * (-(-E // (NW * B * KCI)))
TOTB = NW * K
EPAD = TOTB * B
ROWS_PT = NPAD // NS
R = 1024
KA = 48
KB = 112


def _zero_rows(buf, nrows):
    def zi(i, c):
        def zj(j, c2):
            buf[i, pl.ds(j * 16, 16)] = jnp.zeros((16,), jnp.float32)
            return c2
        return lax.fori_loop(0, D // 16, zj, c)
    lax.fori_loop(0, nrows, zi, 0)


def _spread_zero(src_buf, dst_sh, base):
    nfull = ROWS_PT // B
    rem = ROWS_PT - nfull * B
    for t in range(nfull):
        pltpu.sync_copy(src_buf, dst_sh.at[pl.ds(base + t * B, B)])
    if rem:
        pltpu.sync_copy(src_buf.at[pl.ds(0, rem)],
                        dst_sh.at[pl.ds(base + nfull * B, rem)])


def _sc_aggregate(y, ij):
    out_types = (jax.ShapeDtypeStruct((NC, NPAD, D), jnp.float32),)
    scratch = [
        pltpu.VMEM_SHARED((NPAD, D), jnp.float32),
        pltpu.VMEM((KCI, 2, B), jnp.int32),
        pltpu.VMEM((2, B, D), jnp.float32),
        pltpu.SemaphoreType.DMA,
        pltpu.SemaphoreType.DMA,
    ]
    mesh = plsc.VectorSubcoreMesh(core_axis_name="c", subcore_axis_name="s")

    def body(y_hbm, ij_hbm, part_hbm, acc_sh, idx_v, rows_v, s0, s1):
        cid = lax.axis_index("c")
        sid = lax.axis_index("s")
        base = sid * ROWS_PT
        sems = (s0, s1)
        start = jnp.where(cid == 0, sid * KA, 16 * KA + sid * KB)
        nchunks = jnp.where(cid == 0, KA // KCI, KB // KCI)

        _zero_rows(rows_v.at[0], B)
        _spread_zero(rows_v.at[0], acc_sh, base)
        plsc.subcore_barrier()

        def chunk(o, c):
            pltpu.sync_copy(ij_hbm.at[pl.ds(start + o * KCI, KCI)], idx_v)
            cps = [None, None]
            cps[0] = pltpu.async_copy(
                y_hbm.at[idx_v.at[0, 0]], rows_v.at[0], sems[0])
            for k in range(KCI):
                p = k % 2
                cps[p].wait()
                if k + 1 < KCI:
                    cps[1 - p] = pltpu.async_copy(
                        y_hbm.at[idx_v.at[k + 1, 0]], rows_v.at[1 - p],
                        sems[1 - p])
                pltpu.sync_copy(rows_v.at[p], acc_sh.at[idx_v.at[k, 1]],
                                add=True)
            return c
        lax.fori_loop(0, nchunks, chunk, 0)

        plsc.subcore_barrier()
        pltpu.sync_copy(acc_sh.at[pl.ds(base, ROWS_PT)],
                        part_hbm.at[cid, pl.ds(base, ROWS_PT)])

    fn = pl.kernel(body, out_type=out_types, mesh=mesh,
                   scratch_types=scratch)
    return fn(y, ij)[0]


def _sc_degree(ij):
    out_types = (jax.ShapeDtypeStruct((NC, NPAD, D), jnp.float32),)
    scratch = [
        pltpu.VMEM_SHARED((NPAD, D), jnp.float32),
        pltpu.VMEM((KC, 2, B), jnp.int32),
        pltpu.VMEM((B, D), jnp.float32),
    ]
    mesh = plsc.VectorSubcoreMesh(core_axis_name="c", subcore_axis_name="s")

    def body(ij_hbm, degp_hbm, deg_sh, idx_v, ones_v):
        cid = lax.axis_index("c")
        sid = lax.axis_index("s")
        wid = sid * NC + cid
        wstart = wid * K
        base = sid * ROWS_PT

        _zero_rows(ones_v, B)
        _spread_zero(ones_v, deg_sh, base)

        def one_i(i, c):
            def one_j(j, c2):
                ones_v[i, pl.ds(j * 16, 16)] = jnp.ones((16,), jnp.float32)
                return c2
            return lax.fori_loop(0, D // 16, one_j, c)
        lax.fori_loop(0, B, one_i, 0)
        plsc.subcore_barrier()

        def chunk(o, c):
            pltpu.sync_copy(ij_hbm.at[pl.ds(wstart + o * KC, KC)], idx_v)

            def edge_batch(j, c2):
                pltpu.sync_copy(ones_v, deg_sh.at[idx_v.at[j, 1]], add=True)
                return c2
            return lax.fori_loop(0, KC, edge_batch, c)
        lax.fori_loop(0, K // KC, chunk, 0)

        plsc.subcore_barrier()
        pltpu.sync_copy(deg_sh.at[pl.ds(base, ROWS_PT)],
                        degp_hbm.at[cid, pl.ds(base, ROWS_PT)])

    fn = pl.kernel(body, out_type=out_types, mesh=mesh,
                   scratch_types=scratch)
    return fn(ij)[0]


def _tc_pre(x, ws, wn, b):
    def body(x_ref, ws_ref, wn_ref, b_ref, s_ref, y_ref):
        xb = x_ref[...]
        s_ref[...] = jnp.dot(xb, ws_ref[...],
                             preferred_element_type=jnp.float32) + b_ref[...]
        y_ref[...] = jnp.dot(xb, wn_ref[...],
                             preferred_element_type=jnp.float32)
    return pl.pallas_call(
        body,
        grid=(NPAD // R,),
        in_specs=[
            pl.BlockSpec((R, D), lambda i: (i, 0)),
            pl.BlockSpec((D, D), lambda i: (0, 0)),
            pl.BlockSpec((D, D), lambda i: (0, 0)),
            pl.BlockSpec((1, D), lambda i: (0, 0)),
        ],
        out_specs=[pl.BlockSpec((R, D), lambda i: (i, 0)),
                   pl.BlockSpec((R, D), lambda i: (i, 0))],
        out_shape=[jax.ShapeDtypeStruct((NPAD, D), jnp.float32)] * 2,
    )(x, ws, wn, b)


def _tc_mid(s1, parts, degp, ws, wn, b):
    def body(s1_ref, p_ref, dg_ref, ws_ref, wn_ref, b_ref, s2_ref, y2_ref):
        agg = p_ref[0] + p_ref[1]
        deg = dg_ref[0, :, 0:1] + dg_ref[1, :, 0:1]
        rdeg = 1.0 / jnp.maximum(deg, 1.0)
        h1 = jnp.maximum(s1_ref[...] + agg * rdeg, 0.0)
        s2_ref[...] = jnp.dot(h1, ws_ref[...],
                              preferred_element_type=jnp.float32) + b_ref[...]
        y2_ref[...] = jnp.dot(h1, wn_ref[...],
                              preferred_element_type=jnp.float32)
    return pl.pallas_call(
        body,
        grid=(NPAD // R,),
        in_specs=[
            pl.BlockSpec((R, D), lambda i: (i, 0)),
            pl.BlockSpec((NC, R, D), lambda i: (0, i, 0)),
            pl.BlockSpec((NC, R, D), lambda i: (0, i, 0)),
            pl.BlockSpec((D, D), lambda i: (0, 0)),
            pl.BlockSpec((D, D), lambda i: (0, 0)),
            pl.BlockSpec((1, D), lambda i: (0, 0)),
        ],
        out_specs=[pl.BlockSpec((R, D), lambda i: (i, 0)),
                   pl.BlockSpec((R, D), lambda i: (i, 0))],
        out_shape=[jax.ShapeDtypeStruct((NPAD, D), jnp.float32)] * 2,
    )(s1, parts, degp, ws, wn, b)


def _tc_post(s2, parts, degp):
    def body(s2_ref, p_ref, dg_ref, o_ref):
        agg = p_ref[0] + p_ref[1]
        deg = dg_ref[0, :, 0:1] + dg_ref[1, :, 0:1]
        o_ref[...] = s2_ref[...] + agg * (1.0 / jnp.maximum(deg, 1.0))
    return pl.pallas_call(
        body,
        grid=(NPAD // R,),
        in_specs=[
            pl.BlockSpec((R, D), lambda i: (i, 0)),
            pl.BlockSpec((NC, R, D), lambda i: (0, i, 0)),
            pl.BlockSpec((NC, R, D), lambda i: (0, i, 0)),
        ],
        out_specs=pl.BlockSpec((R, D), lambda i: (i, 0)),
        out_shape=jax.ShapeDtypeStruct((NPAD, D), jnp.float32),
    )(s2, parts, degp)


def kernel(in_feat, edge_index, W1_self, W1_neigh, b1, W2_self, W2_neigh, b2):
    src = edge_index[0].astype(jnp.int32)
    dst = edge_index[1].astype(jnp.int32)
    pad = EPAD - E
    srcp = jnp.concatenate([src, jnp.zeros((pad,), jnp.int32)]).reshape(TOTB, B)
    dstp = jnp.concatenate([dst, jnp.full((pad,), N, jnp.int32)]).reshape(TOTB, B)
    ij = jnp.stack([srcp, dstp], axis=1)
    x = jnp.pad(in_feat, ((0, NPAD - N), (0, 0)))

    degp = _sc_degree(ij)
    s1, y1 = _tc_pre(x, W1_self, W1_neigh, b1.reshape(1, D))
    parts1 = _sc_aggregate(y1, ij)
    s2, y2 = _tc_mid(s1, parts1, degp, W2_self, W2_neigh, b2.reshape(1, D))
    parts2 = _sc_aggregate(y2, ij)
    out = _tc_post(s2, parts2, degp)
    return out[:N]


# speedup vs baseline: 1.1279x; 1.1279x over previous
import jax
import jax.numpy as jnp
from jax import lax
from jax.experimental import pallas as pl
from jax.experimental.pallas import tpu as pltpu
from jax.experimental.pallas import tpu_sc as plsc

N = 10000
E = 320000
D = 128
NPAD = 10240
NC = 2
NS = 16
NW = NC * NS
B = 128
KCI = 16
KC = 8
K = KCI * (-(-E // (NW * B * KCI)))
TOTB = NW * K
EPAD = TOTB * B
ROWS_PT = NPAD // NS
R = 1024
KA = 112
KB = 48


def _zero_rows(buf, nrows):
    def zi(i, c):
        def zj(j, c2):
            buf[i, pl.ds(j * 16, 16)] = jnp.zeros((16,), jnp.float32)
            return c2
        return lax.fori_loop(0, D // 16, zj, c)
    lax.fori_loop(0, nrows, zi, 0)


def _spread_zero(src_buf, dst_sh, base):
    nfull = ROWS_PT // B
    rem = ROWS_PT - nfull * B
    for t in range(nfull):
        pltpu.sync_copy(src_buf, dst_sh.at[pl.ds(base + t * B, B)])
    if rem:
        pltpu.sync_copy(src_buf.at[pl.ds(0, rem)],
                        dst_sh.at[pl.ds(base + nfull * B, rem)])


def _sc_aggregate(y, ij):
    out_types = (jax.ShapeDtypeStruct((NC, NPAD, D), jnp.float32),)
    scratch = [
        pltpu.VMEM_SHARED((NPAD, D), jnp.float32),
        pltpu.VMEM((KCI, 2, B), jnp.int32),
        pltpu.VMEM((2, B, D), jnp.float32),
        pltpu.SemaphoreType.DMA,
        pltpu.SemaphoreType.DMA,
    ]
    mesh = plsc.VectorSubcoreMesh(core_axis_name="c", subcore_axis_name="s")

    def body(y_hbm, ij_hbm, part_hbm, acc_sh, idx_v, rows_v, s0, s1):
        cid = lax.axis_index("c")
        sid = lax.axis_index("s")
        base = sid * ROWS_PT
        sems = (s0, s1)
        start = jnp.where(cid == 0, sid * KA, 16 * KA + sid * KB)
        nchunks = jnp.where(cid == 0, KA // KCI, KB // KCI)

        _zero_rows(rows_v.at[0], B)
        _spread_zero(rows_v.at[0], acc_sh, base)
        plsc.subcore_barrier()

        def chunk(o, c):
            pltpu.sync_copy(ij_hbm.at[pl.ds(start + o * KCI, KCI)], idx_v)
            cps = [None, None]
            cps[0] = pltpu.async_copy(
                y_hbm.at[idx_v.at[0, 0]], rows_v.at[0], sems[0])
            for k in range(KCI):
                p = k % 2
                cps[p].wait()
                if k + 1 < KCI:
                    cps[1 - p] = pltpu.async_copy(
                        y_hbm.at[idx_v.at[k + 1, 0]], rows_v.at[1 - p],
                        sems[1 - p])
                pltpu.sync_copy(rows_v.at[p], acc_sh.at[idx_v.at[k, 1]],
                                add=True)
            return c
        lax.fori_loop(0, nchunks, chunk, 0)

        plsc.subcore_barrier()
        pltpu.sync_copy(acc_sh.at[pl.ds(base, ROWS_PT)],
                        part_hbm.at[cid, pl.ds(base, ROWS_PT)])

    fn = pl.kernel(body, out_type=out_types, mesh=mesh,
                   scratch_types=scratch)
    return fn(y, ij)[0]


def _sc_degree(ij):
    out_types = (jax.ShapeDtypeStruct((NC, NPAD, D), jnp.float32),)
    scratch = [
        pltpu.VMEM_SHARED((NPAD, D), jnp.float32),
        pltpu.VMEM((KC, 2, B), jnp.int32),
        pltpu.VMEM((B, D), jnp.float32),
    ]
    mesh = plsc.VectorSubcoreMesh(core_axis_name="c", subcore_axis_name="s")

    def body(ij_hbm, degp_hbm, deg_sh, idx_v, ones_v):
        cid = lax.axis_index("c")
        sid = lax.axis_index("s")
        wid = sid * NC + cid
        wstart = wid * K
        base = sid * ROWS_PT

        _zero_rows(ones_v, B)
        _spread_zero(ones_v, deg_sh, base)

        def one_i(i, c):
            def one_j(j, c2):
                ones_v[i, pl.ds(j * 16, 16)] = jnp.ones((16,), jnp.float32)
                return c2
            return lax.fori_loop(0, D // 16, one_j, c)
        lax.fori_loop(0, B, one_i, 0)
        plsc.subcore_barrier()

        def chunk(o, c):
            pltpu.sync_copy(ij_hbm.at[pl.ds(wstart + o * KC, KC)], idx_v)

            def edge_batch(j, c2):
                pltpu.sync_copy(ones_v, deg_sh.at[idx_v.at[j, 1]], add=True)
                return c2
            return lax.fori_loop(0, KC, edge_batch, c)
        lax.fori_loop(0, K // KC, chunk, 0)

        plsc.subcore_barrier()
        pltpu.sync_copy(deg_sh.at[pl.ds(base, ROWS_PT)],
                        degp_hbm.at[cid, pl.ds(base, ROWS_PT)])

    fn = pl.kernel(body, out_type=out_types, mesh=mesh,
                   scratch_types=scratch)
    return fn(ij)[0]


def _tc_pre(x, ws, wn, b):
    def body(x_ref, ws_ref, wn_ref, b_ref, s_ref, y_ref):
        xb = x_ref[...]
        s_ref[...] = jnp.dot(xb, ws_ref[...],
                             preferred_element_type=jnp.float32) + b_ref[...]
        y_ref[...] = jnp.dot(xb, wn_ref[...],
                             preferred_element_type=jnp.float32)
    return pl.pallas_call(
        body,
        grid=(NPAD // R,),
        in_specs=[
            pl.BlockSpec((R, D), lambda i: (i, 0)),
            pl.BlockSpec((D, D), lambda i: (0, 0)),
            pl.BlockSpec((D, D), lambda i: (0, 0)),
            pl.BlockSpec((1, D), lambda i: (0, 0)),
        ],
        out_specs=[pl.BlockSpec((R, D), lambda i: (i, 0)),
                   pl.BlockSpec((R, D), lambda i: (i, 0))],
        out_shape=[jax.ShapeDtypeStruct((NPAD, D), jnp.float32)] * 2,
    )(x, ws, wn, b)


def _tc_mid(s1, parts, degp, ws, wn, b):
    def body(s1_ref, p_ref, dg_ref, ws_ref, wn_ref, b_ref, s2_ref, y2_ref):
        agg = p_ref[0] + p_ref[1]
        deg = dg_ref[0, :, 0:1] + dg_ref[1, :, 0:1]
        rdeg = 1.0 / jnp.maximum(deg, 1.0)
        h1 = jnp.maximum(s1_ref[...] + agg * rdeg, 0.0)
        s2_ref[...] = jnp.dot(h1, ws_ref[...],
                              preferred_element_type=jnp.float32) + b_ref[...]
        y2_ref[...] = jnp.dot(h1, wn_ref[...],
                              preferred_element_type=jnp.float32)
    return pl.pallas_call(
        body,
        grid=(NPAD // R,),
        in_specs=[
            pl.BlockSpec((R, D), lambda i: (i, 0)),
            pl.BlockSpec((NC, R, D), lambda i: (0, i, 0)),
            pl.BlockSpec((NC, R, D), lambda i: (0, i, 0)),
            pl.BlockSpec((D, D), lambda i: (0, 0)),
            pl.BlockSpec((D, D), lambda i: (0, 0)),
            pl.BlockSpec((1, D), lambda i: (0, 0)),
        ],
        out_specs=[pl.BlockSpec((R, D), lambda i: (i, 0)),
                   pl.BlockSpec((R, D), lambda i: (i, 0))],
        out_shape=[jax.ShapeDtypeStruct((NPAD, D), jnp.float32)] * 2,
    )(s1, parts, degp, ws, wn, b)


def _tc_post(s2, parts, degp):
    def body(s2_ref, p_ref, dg_ref, o_ref):
        agg = p_ref[0] + p_ref[1]
        deg = dg_ref[0, :, 0:1] + dg_ref[1, :, 0:1]
        o_ref[...] = s2_ref[...] + agg * (1.0 / jnp.maximum(deg, 1.0))
    return pl.pallas_call(
        body,
        grid=(NPAD // R,),
        in_specs=[
            pl.BlockSpec((R, D), lambda i: (i, 0)),
            pl.BlockSpec((NC, R, D), lambda i: (0, i, 0)),
            pl.BlockSpec((NC, R, D), lambda i: (0, i, 0)),
        ],
        out_specs=pl.BlockSpec((R, D), lambda i: (i, 0)),
        out_shape=jax.ShapeDtypeStruct((NPAD, D), jnp.float32),
    )(s2, parts, degp)


def kernel(in_feat, edge_index, W1_self, W1_neigh, b1, W2_self, W2_neigh, b2):
    src = edge_index[0].astype(jnp.int32)
    dst = edge_index[1].astype(jnp.int32)
    pad = EPAD - E
    srcp = jnp.concatenate([src, jnp.zeros((pad,), jnp.int32)]).reshape(TOTB, B)
    dstp = jnp.concatenate([dst, jnp.full((pad,), N, jnp.int32)]).reshape(TOTB, B)
    ij = jnp.stack([srcp, dstp], axis=1)
    x = jnp.pad(in_feat, ((0, NPAD - N), (0, 0)))

    degp = _sc_degree(ij)
    s1, y1 = _tc_pre(x, W1_self, W1_neigh, b1.reshape(1, D))
    parts1 = _sc_aggregate(y1, ij)
    s2, y2 = _tc_mid(s1, parts1, degp, W2_self, W2_neigh, b2.reshape(1, D))
    parts2 = _sc_aggregate(y2, ij)
    out = _tc_post(s2, parts2, degp)
    return out[:N]


# speedup vs baseline: 1.1589x; 1.0275x over previous
import jax
import jax.numpy as jnp
from jax import lax
from jax.experimental import pallas as pl
from jax.experimental.pallas import tpu as pltpu
from jax.experimental.pallas import tpu_sc as plsc

N = 10000
E = 320000
D = 128
NPAD = 10240
NC = 2
NS = 16
NW = NC * NS
B = 128
KCI = 16
KC = 8
K = KCI * (-(-E // (NW * B * KCI)))
TOTB = NW * K
EPAD = TOTB * B
ROWS_PT = NPAD // NS
R = 1024
KA = 128
KB = 32


def _zero_rows(buf, nrows):
    def zi(i, c):
        def zj(j, c2):
            buf[i, pl.ds(j * 16, 16)] = jnp.zeros((16,), jnp.float32)
            return c2
        return lax.fori_loop(0, D // 16, zj, c)
    lax.fori_loop(0, nrows, zi, 0)


def _spread_zero(src_buf, dst_sh, base):
    nfull = ROWS_PT // B
    rem = ROWS_PT - nfull * B
    for t in range(nfull):
        pltpu.sync_copy(src_buf, dst_sh.at[pl.ds(base + t * B, B)])
    if rem:
        pltpu.sync_copy(src_buf.at[pl.ds(0, rem)],
                        dst_sh.at[pl.ds(base + nfull * B, rem)])


def _sc_aggregate(y, ij):
    out_types = (jax.ShapeDtypeStruct((NC, NPAD, D), jnp.float32),)
    scratch = [
        pltpu.VMEM_SHARED((NPAD, D), jnp.float32),
        pltpu.VMEM((KCI, 2, B), jnp.int32),
        pltpu.VMEM((2, B, D), jnp.float32),
        pltpu.SemaphoreType.DMA,
        pltpu.SemaphoreType.DMA,
    ]
    mesh = plsc.VectorSubcoreMesh(core_axis_name="c", subcore_axis_name="s")

    def body(y_hbm, ij_hbm, part_hbm, acc_sh, idx_v, rows_v, s0, s1):
        cid = lax.axis_index("c")
        sid = lax.axis_index("s")
        base = sid * ROWS_PT
        sems = (s0, s1)
        start = jnp.where(cid == 0, sid * KA, 16 * KA + sid * KB)
        nchunks = jnp.where(cid == 0, KA // KCI, KB // KCI)

        _zero_rows(rows_v.at[0], B)
        _spread_zero(rows_v.at[0], acc_sh, base)
        plsc.subcore_barrier()

        def chunk(o, c):
            pltpu.sync_copy(ij_hbm.at[pl.ds(start + o * KCI, KCI)], idx_v)
            cps = [None, None]
            cps[0] = pltpu.async_copy(
                y_hbm.at[idx_v.at[0, 0]], rows_v.at[0], sems[0])
            for k in range(KCI):
                p = k % 2
                cps[p].wait()
                if k + 1 < KCI:
                    cps[1 - p] = pltpu.async_copy(
                        y_hbm.at[idx_v.at[k + 1, 0]], rows_v.at[1 - p],
                        sems[1 - p])
                pltpu.sync_copy(rows_v.at[p], acc_sh.at[idx_v.at[k, 1]],
                                add=True)
            return c
        lax.fori_loop(0, nchunks, chunk, 0)

        plsc.subcore_barrier()
        pltpu.sync_copy(acc_sh.at[pl.ds(base, ROWS_PT)],
                        part_hbm.at[cid, pl.ds(base, ROWS_PT)])

    fn = pl.kernel(body, out_type=out_types, mesh=mesh,
                   scratch_types=scratch)
    return fn(y, ij)[0]


def _sc_degree(ij):
    out_types = (jax.ShapeDtypeStruct((NC, NPAD, D), jnp.float32),)
    scratch = [
        pltpu.VMEM_SHARED((NPAD, D), jnp.float32),
        pltpu.VMEM((KC, 2, B), jnp.int32),
        pltpu.VMEM((B, D), jnp.float32),
    ]
    mesh = plsc.VectorSubcoreMesh(core_axis_name="c", subcore_axis_name="s")

    def body(ij_hbm, degp_hbm, deg_sh, idx_v, ones_v):
        cid = lax.axis_index("c")
        sid = lax.axis_index("s")
        wid = sid * NC + cid
        wstart = wid * K
        base = sid * ROWS_PT

        _zero_rows(ones_v, B)
        _spread_zero(ones_v, deg_sh, base)

        def one_i(i, c):
            def one_j(j, c2):
                ones_v[i, pl.ds(j * 16, 16)] = jnp.ones((16,), jnp.float32)
                return c2
            return lax.fori_loop(0, D // 16, one_j, c)
        lax.fori_loop(0, B, one_i, 0)
        plsc.subcore_barrier()

        def chunk(o, c):
            pltpu.sync_copy(ij_hbm.at[pl.ds(wstart + o * KC, KC)], idx_v)

            def edge_batch(j, c2):
                pltpu.sync_copy(ones_v, deg_sh.at[idx_v.at[j, 1]], add=True)
                return c2
            return lax.fori_loop(0, KC, edge_batch, c)
        lax.fori_loop(0, K // KC, chunk, 0)

        plsc.subcore_barrier()
        pltpu.sync_copy(deg_sh.at[pl.ds(base, ROWS_PT)],
                        degp_hbm.at[cid, pl.ds(base, ROWS_PT)])

    fn = pl.kernel(body, out_type=out_types, mesh=mesh,
                   scratch_types=scratch)
    return fn(ij)[0]


def _tc_pre(x, ws, wn, b):
    def body(x_ref, ws_ref, wn_ref, b_ref, s_ref, y_ref):
        xb = x_ref[...]
        s_ref[...] = jnp.dot(xb, ws_ref[...],
                             preferred_element_type=jnp.float32) + b_ref[...]
        y_ref[...] = jnp.dot(xb, wn_ref[...],
                             preferred_element_type=jnp.float32)
    return pl.pallas_call(
        body,
        grid=(NPAD // R,),
        in_specs=[
            pl.BlockSpec((R, D), lambda i: (i, 0)),
            pl.BlockSpec((D, D), lambda i: (0, 0)),
            pl.BlockSpec((D, D), lambda i: (0, 0)),
            pl.BlockSpec((1, D), lambda i: (0, 0)),
        ],
        out_specs=[pl.BlockSpec((R, D), lambda i: (i, 0)),
                   pl.BlockSpec((R, D), lambda i: (i, 0))],
        out_shape=[jax.ShapeDtypeStruct((NPAD, D), jnp.float32)] * 2,
    )(x, ws, wn, b)


def _tc_mid(s1, parts, degp, ws, wn, b):
    def body(s1_ref, p_ref, dg_ref, ws_ref, wn_ref, b_ref, s2_ref, y2_ref):
        agg = p_ref[0] + p_ref[1]
        deg = dg_ref[0, :, 0:1] + dg_ref[1, :, 0:1]
        rdeg = 1.0 / jnp.maximum(deg, 1.0)
        h1 = jnp.maximum(s1_ref[...] + agg * rdeg, 0.0)
        s2_ref[...] = jnp.dot(h1, ws_ref[...],
                              preferred_element_type=jnp.float32) + b_ref[...]
        y2_ref[...] = jnp.dot(h1, wn_ref[...],
                              preferred_element_type=jnp.float32)
    return pl.pallas_call(
        body,
        grid=(NPAD // R,),
        in_specs=[
            pl.BlockSpec((R, D), lambda i: (i, 0)),
            pl.BlockSpec((NC, R, D), lambda i: (0, i, 0)),
            pl.BlockSpec((NC, R, D), lambda i: (0, i, 0)),
            pl.BlockSpec((D, D), lambda i: (0, 0)),
            pl.BlockSpec((D, D), lambda i: (0, 0)),
            pl.BlockSpec((1, D), lambda i: (0, 0)),
        ],
        out_specs=[pl.BlockSpec((R, D), lambda i: (i, 0)),
                   pl.BlockSpec((R, D), lambda i: (i, 0))],
        out_shape=[jax.ShapeDtypeStruct((NPAD, D), jnp.float32)] * 2,
    )(s1, parts, degp, ws, wn, b)


def _tc_post(s2, parts, degp):
    def body(s2_ref, p_ref, dg_ref, o_ref):
        agg = p_ref[0] + p_ref[1]
        deg = dg_ref[0, :, 0:1] + dg_ref[1, :, 0:1]
        o_ref[...] = s2_ref[...] + agg * (1.0 / jnp.maximum(deg, 1.0))
    return pl.pallas_call(
        body,
        grid=(NPAD // R,),
        in_specs=[
            pl.BlockSpec((R, D), lambda i: (i, 0)),
            pl.BlockSpec((NC, R, D), lambda i: (0, i, 0)),
            pl.BlockSpec((NC, R, D), lambda i: (0, i, 0)),
        ],
        out_specs=pl.BlockSpec((R, D), lambda i: (i, 0)),
        out_shape=jax.ShapeDtypeStruct((NPAD, D), jnp.float32),
    )(s2, parts, degp)


def kernel(in_feat, edge_index, W1_self, W1_neigh, b1, W2_self, W2_neigh, b2):
    src = edge_index[0].astype(jnp.int32)
    dst = edge_index[1].astype(jnp.int32)
    pad = EPAD - E
    srcp = jnp.concatenate([src, jnp.zeros((pad,), jnp.int32)]).reshape(TOTB, B)
    dstp = jnp.concatenate([dst, jnp.full((pad,), N, jnp.int32)]).reshape(TOTB, B)
    ij = jnp.stack([srcp, dstp], axis=1)
    x = jnp.pad(in_feat, ((0, NPAD - N), (0, 0)))

    degp = _sc_degree(ij)
    s1, y1 = _tc_pre(x, W1_self, W1_neigh, b1.reshape(1, D))
    parts1 = _sc_aggregate(y1, ij)
    s2, y2 = _tc_mid(s1, parts1, degp, W2_self, W2_neigh, b2.reshape(1, D))
    parts2 = _sc_aggregate(y2, ij)
    out = _tc_post(s2, parts2, degp)
    return out[:N]


# speedup vs baseline: 1.2041x; 1.0390x over previous
import jax
import jax.numpy as jnp
from jax import lax
from jax.experimental import pallas as pl
from jax.experimental.pallas import tpu as pltpu
from jax.experimental.pallas import tpu_sc as plsc

N = 10000
E = 320000
D = 128
NPAD = 10240
NC = 2
NS = 16
NW = NC * NS
B = 128
KCI = 16
KC = 8
K = KCI * (-(-E // (NW * B * KCI)))
TOTB = NW * K
EPAD = TOTB * B
ROWS_PT = NPAD // NS
R = 1024
KA = 144
KB = 16


def _zero_rows(buf, nrows):
    def zi(i, c):
        def zj(j, c2):
            buf[i, pl.ds(j * 16, 16)] = jnp.zeros((16,), jnp.float32)
            return c2
        return lax.fori_loop(0, D // 16, zj, c)
    lax.fori_loop(0, nrows, zi, 0)


def _spread_zero(src_buf, dst_sh, base):
    nfull = ROWS_PT // B
    rem = ROWS_PT - nfull * B
    for t in range(nfull):
        pltpu.sync_copy(src_buf, dst_sh.at[pl.ds(base + t * B, B)])
    if rem:
        pltpu.sync_copy(src_buf.at[pl.ds(0, rem)],
                        dst_sh.at[pl.ds(base + nfull * B, rem)])


def _sc_aggregate(y, ij):
    out_types = (jax.ShapeDtypeStruct((NC, NPAD, D), jnp.float32),)
    scratch = [
        pltpu.VMEM_SHARED((NPAD, D), jnp.float32),
        pltpu.VMEM((KCI, 2, B), jnp.int32),
        pltpu.VMEM((2, B, D), jnp.float32),
        pltpu.SemaphoreType.DMA,
        pltpu.SemaphoreType.DMA,
    ]
    mesh = plsc.VectorSubcoreMesh(core_axis_name="c", subcore_axis_name="s")

    def body(y_hbm, ij_hbm, part_hbm, acc_sh, idx_v, rows_v, s0, s1):
        cid = lax.axis_index("c")
        sid = lax.axis_index("s")
        base = sid * ROWS_PT
        sems = (s0, s1)
        start = jnp.where(cid == 0, sid * KA, 16 * KA + sid * KB)
        nchunks = jnp.where(cid == 0, KA // KCI, KB // KCI)

        _zero_rows(rows_v.at[0], B)
        _spread_zero(rows_v.at[0], acc_sh, base)
        plsc.subcore_barrier()

        def chunk(o, c):
            pltpu.sync_copy(ij_hbm.at[pl.ds(start + o * KCI, KCI)], idx_v)
            cps = [None, None]
            cps[0] = pltpu.async_copy(
                y_hbm.at[idx_v.at[0, 0]], rows_v.at[0], sems[0])
            for k in range(KCI):
                p = k % 2
                cps[p].wait()
                if k + 1 < KCI:
                    cps[1 - p] = pltpu.async_copy(
                        y_hbm.at[idx_v.at[k + 1, 0]], rows_v.at[1 - p],
                        sems[1 - p])
                pltpu.sync_copy(rows_v.at[p], acc_sh.at[idx_v.at[k, 1]],
                                add=True)
            return c
        lax.fori_loop(0, nchunks, chunk, 0)

        plsc.subcore_barrier()
        pltpu.sync_copy(acc_sh.at[pl.ds(base, ROWS_PT)],
                        part_hbm.at[cid, pl.ds(base, ROWS_PT)])

    fn = pl.kernel(body, out_type=out_types, mesh=mesh,
                   scratch_types=scratch)
    return fn(y, ij)[0]


def _sc_degree(ij):
    out_types = (jax.ShapeDtypeStruct((NC, NPAD, D), jnp.float32),)
    scratch = [
        pltpu.VMEM_SHARED((NPAD, D), jnp.float32),
        pltpu.VMEM((KC, 2, B), jnp.int32),
        pltpu.VMEM((B, D), jnp.float32),
    ]
    mesh = plsc.VectorSubcoreMesh(core_axis_name="c", subcore_axis_name="s")

    def body(ij_hbm, degp_hbm, deg_sh, idx_v, ones_v):
        cid = lax.axis_index("c")
        sid = lax.axis_index("s")
        wid = sid * NC + cid
        wstart = wid * K
        base = sid * ROWS_PT

        _zero_rows(ones_v, B)
        _spread_zero(ones_v, deg_sh, base)

        def one_i(i, c):
            def one_j(j, c2):
                ones_v[i, pl.ds(j * 16, 16)] = jnp.ones((16,), jnp.float32)
                return c2
            return lax.fori_loop(0, D // 16, one_j, c)
        lax.fori_loop(0, B, one_i, 0)
        plsc.subcore_barrier()

        def chunk(o, c):
            pltpu.sync_copy(ij_hbm.at[pl.ds(wstart + o * KC, KC)], idx_v)

            def edge_batch(j, c2):
                pltpu.sync_copy(ones_v, deg_sh.at[idx_v.at[j, 1]], add=True)
                return c2
            return lax.fori_loop(0, KC, edge_batch, c)
        lax.fori_loop(0, K // KC, chunk, 0)

        plsc.subcore_barrier()
        pltpu.sync_copy(deg_sh.at[pl.ds(base, ROWS_PT)],
                        degp_hbm.at[cid, pl.ds(base, ROWS_PT)])

    fn = pl.kernel(body, out_type=out_types, mesh=mesh,
                   scratch_types=scratch)
    return fn(ij)[0]


def _tc_pre(x, ws, wn, b):
    def body(x_ref, ws_ref, wn_ref, b_ref, s_ref, y_ref):
        xb = x_ref[...]
        s_ref[...] = jnp.dot(xb, ws_ref[...],
                             preferred_element_type=jnp.float32) + b_ref[...]
        y_ref[...] = jnp.dot(xb, wn_ref[...],
                             preferred_element_type=jnp.float32)
    return pl.pallas_call(
        body,
        grid=(NPAD // R,),
        in_specs=[
            pl.BlockSpec((R, D), lambda i: (i, 0)),
            pl.BlockSpec((D, D), lambda i: (0, 0)),
            pl.BlockSpec((D, D), lambda i: (0, 0)),
            pl.BlockSpec((1, D), lambda i: (0, 0)),
        ],
        out_specs=[pl.BlockSpec((R, D), lambda i: (i, 0)),
                   pl.BlockSpec((R, D), lambda i: (i, 0))],
        out_shape=[jax.ShapeDtypeStruct((NPAD, D), jnp.float32)] * 2,
    )(x, ws, wn, b)


def _tc_mid(s1, parts, degp, ws, wn, b):
    def body(s1_ref, p_ref, dg_ref, ws_ref, wn_ref, b_ref, s2_ref, y2_ref):
        agg = p_ref[0] + p_ref[1]
        deg = dg_ref[0, :, 0:1] + dg_ref[1, :, 0:1]
        rdeg = 1.0 / jnp.maximum(deg, 1.0)
        h1 = jnp.maximum(s1_ref[...] + agg * rdeg, 0.0)
        s2_ref[...] = jnp.dot(h1, ws_ref[...],
                              preferred_element_type=jnp.float32) + b_ref[...]
        y2_ref[...] = jnp.dot(h1, wn_ref[...],
                              preferred_element_type=jnp.float32)
    return pl.pallas_call(
        body,
        grid=(NPAD // R,),
        in_specs=[
            pl.BlockSpec((R, D), lambda i: (i, 0)),
            pl.BlockSpec((NC, R, D), lambda i: (0, i, 0)),
            pl.BlockSpec((NC, R, D), lambda i: (0, i, 0)),
            pl.BlockSpec((D, D), lambda i: (0, 0)),
            pl.BlockSpec((D, D), lambda i: (0, 0)),
            pl.BlockSpec((1, D), lambda i: (0, 0)),
        ],
        out_specs=[pl.BlockSpec((R, D), lambda i: (i, 0)),
                   pl.BlockSpec((R, D), lambda i: (i, 0))],
        out_shape=[jax.ShapeDtypeStruct((NPAD, D), jnp.float32)] * 2,
    )(s1, parts, degp, ws, wn, b)


def _tc_post(s2, parts, degp):
    def body(s2_ref, p_ref, dg_ref, o_ref):
        agg = p_ref[0] + p_ref[1]
        deg = dg_ref[0, :, 0:1] + dg_ref[1, :, 0:1]
        o_ref[...] = s2_ref[...] + agg * (1.0 / jnp.maximum(deg, 1.0))
    return pl.pallas_call(
        body,
        grid=(NPAD // R,),
        in_specs=[
            pl.BlockSpec((R, D), lambda i: (i, 0)),
            pl.BlockSpec((NC, R, D), lambda i: (0, i, 0)),
            pl.BlockSpec((NC, R, D), lambda i: (0, i, 0)),
        ],
        out_specs=pl.BlockSpec((R, D), lambda i: (i, 0)),
        out_shape=jax.ShapeDtypeStruct((NPAD, D), jnp.float32),
    )(s2, parts, degp)


def kernel(in_feat, edge_index, W1_self, W1_neigh, b1, W2_self, W2_neigh, b2):
    src = edge_index[0].astype(jnp.int32)
    dst = edge_index[1].astype(jnp.int32)
    pad = EPAD - E
    srcp = jnp.concatenate([src, jnp.zeros((pad,), jnp.int32)]).reshape(TOTB, B)
    dstp = jnp.concatenate([dst, jnp.full((pad,), N, jnp.int32)]).reshape(TOTB, B)
    ij = jnp.stack([srcp, dstp], axis=1)
    x = jnp.pad(in_feat, ((0, NPAD - N), (0, 0)))

    degp = _sc_degree(ij)
    s1, y1 = _tc_pre(x, W1_self, W1_neigh, b1.reshape(1, D))
    parts1 = _sc_aggregate(y1, ij)
    s2, y2 = _tc_mid(s1, parts1, degp, W2_self, W2_neigh, b2.reshape(1, D))
    parts2 = _sc_aggregate(y2, ij)
    out = _tc_post(s2, parts2, degp)
    return out[:N]
